# baseline (speedup 1.0000x reference)
import functools

import jax
import jax.numpy as jnp
from jax import lax
from jax.experimental import pallas as pl
from jax.experimental.pallas import tpu as pltpu
from jax.experimental.pallas import tpu_sc as plsc

_N = 10000
_E = 320000
_DF = 128
_H = 64
_NCLS = 40

_NSC = 2
_NTILE = 16
_NW = _NSC * _NTILE
_K = 128
_NCHUNK = 80
_EP = _NCHUNK * _K
_EPAD = _NW * _EP
_NPAD = 10240

_RB = 1000
_NRB = _N // _RB


_NBUF = 5
_GAHEAD = 2


def _sc_propagate_body(m_hbm, src_hbm, dst_hbm, w_hbm, out_hbm,
                       src_big, dst_big, w_big, rows_v, acc_sh,
                       gs0, gs1, gs2, gs3, gs4, ss0, ss1, ss2, ss3, ss4):
    gsem = (gs0, gs1, gs2, gs3, gs4)
    ssem = (ss0, ss1, ss2, ss3, ss4)
    c = lax.axis_index("c")
    s = lax.axis_index("s")
    wid = s * _NSC + c

    zero16 = jnp.zeros((16,), jnp.float32)
    for i in range(_K):
        for j in range(_H // 16):
            rows_v[0, i, pl.ds(j * 16, 16)] = zero16
    for z in range(_NPAD // (_NTILE * _K)):
        pltpu.sync_copy(rows_v.at[0],
                        acc_sh.at[pl.ds((z * _NTILE + s) * _K, _K)])

    pltpu.sync_copy(src_hbm.at[wid], src_big)
    pltpu.sync_copy(dst_hbm.at[wid], dst_big)
    pltpu.sync_copy(w_hbm.at[wid], w_big)
    plsc.subcore_barrier()

    def gather(cn, bn):
        pltpu.async_copy(m_hbm.at[src_big.at[cn]], rows_v.at[bn], gsem[bn])

    def wait_gather(cc, b):
        pltpu.make_async_copy(m_hbm.at[src_big.at[cc]], rows_v.at[b],
                              gsem[b]).wait()

    def scatter_start(cc, b):
        pltpu.async_copy(rows_v.at[b], acc_sh.at[dst_big.at[cc]], ssem[b],
                         add=True)

    def wait_scatter(cc, b):
        pltpu.make_async_copy(rows_v.at[b], acc_sh.at[dst_big.at[cc]],
                              ssem[b]).wait()

    def compute(b, cc):
        def grp(g, carry):
            wv = w_big[cc, pl.ds(g * 16, 16)]
            for e16 in range(16):
                wgt = wv[e16]
                e = g * 16 + e16
                for j in range(_H // 16):
                    sl = pl.ds(j * 16, 16)
                    rows_v[b, e, sl] = rows_v[b, e, sl] * wgt
            return carry
        lax.fori_loop(0, _K // 16, grp, 0)

    for b in range(_GAHEAD):
        gather(b, b)

    def quad(t, carry):
        for b in range(_NBUF):
            cc = t * _NBUF + b
            wait_gather(cc, b)
            compute(b, cc)
            scatter_start(cc, b)
            cn = cc + _GAHEAD
            bn = (b + _GAHEAD) % _NBUF

            @pl.when(cn < _NCHUNK)
            def _():
                @pl.when(cn >= _NBUF)
                def _():
                    wait_scatter(cn - _NBUF, bn)
                gather(cn, bn)
        return carry

    lax.fori_loop(0, _NCHUNK // _NBUF, quad, 0)

    for b in range(_NBUF):
        wait_scatter(_NCHUNK - _NBUF + b, b)
    plsc.subcore_barrier()

    for z in range(_NPAD // (_NTILE * _K)):
        base = (z * _NTILE + s) * _K
        pltpu.sync_copy(acc_sh.at[pl.ds(base, _K)],
                        out_hbm.at[c, pl.ds(base, _K)])


@functools.lru_cache(maxsize=None)
def _get_sc_propagate():
    return pl.kernel(
        _sc_propagate_body,
        out_type=jax.ShapeDtypeStruct((_NSC, _NPAD, _H), jnp.float32),
        mesh=plsc.VectorSubcoreMesh(core_axis_name="c", subcore_axis_name="s"),
        compiler_params=pltpu.CompilerParams(use_tc_tiling_on_sc=False),
        scratch_types=(
            [pltpu.VMEM((_NCHUNK, _K), jnp.int32),
             pltpu.VMEM((_NCHUNK, _K), jnp.int32),
             pltpu.VMEM((_NCHUNK, _K), jnp.float32),
             pltpu.VMEM((_NBUF, _K, _H), jnp.float32),
             pltpu.VMEM_SHARED((_NPAD, _H), jnp.float32)]
            + [pltpu.SemaphoreType.DMA] * (2 * _NBUF)
        ),
    )


def _prep_edges(src, dst, w):
    pad = _EPAD - _E
    return (jnp.pad(src, (0, pad)).reshape(_NW, _NCHUNK, _K),
            jnp.pad(dst, (0, pad)).reshape(_NW, _NCHUNK, _K),
            jnp.pad(w, (0, pad)).reshape(_NW, _NCHUNK, _K))


def _propagate(m, src3, dst3, w3):
    return _get_sc_propagate()(m, src3, dst3, w3)


def _proj_mm_body(x_ref, wi_ref, bi_ref, w0_ref, h_ref, m_ref):
    h = jax.nn.relu(
        jnp.dot(x_ref[...], wi_ref[...], preferred_element_type=jnp.float32)
        + bi_ref[...])
    h_ref[...] = h
    m_ref[...] = jnp.dot(h, w0_ref[...], preferred_element_type=jnp.float32)


_proj_mm = pl.pallas_call(
    _proj_mm_body,
    grid=(_NRB,),
    in_specs=[pl.BlockSpec((_RB, _DF), lambda i: (i, 0)),
              pl.BlockSpec((_DF, _H), lambda i: (0, 0)),
              pl.BlockSpec((1, _H), lambda i: (0, 0)),
              pl.BlockSpec((_H, _H), lambda i: (0, 0))],
    out_specs=[pl.BlockSpec((_RB, _H), lambda i: (i, 0)),
               pl.BlockSpec((_RB, _H), lambda i: (i, 0))],
    out_shape=[jax.ShapeDtypeStruct((_N, _H), jnp.float32),
               jax.ShapeDtypeStruct((_N, _H), jnp.float32)],
)


def _gru_math(p_ref, q_ref, h_ref, wih_ref, whh_ref, bih_ref, bhh_ref,
              final_relu):
    h = h_ref[...]
    agg = p_ref[0] + q_ref[0]
    gi = jnp.dot(agg, wih_ref[...],
                 preferred_element_type=jnp.float32) + bih_ref[...]
    gh = jnp.dot(h, whh_ref[...],
                 preferred_element_type=jnp.float32) + bhh_ref[...]
    r = jax.nn.sigmoid(gi[:, 0:_H] + gh[:, 0:_H])
    z = jax.nn.sigmoid(gi[:, _H:2 * _H] + gh[:, _H:2 * _H])
    n = jnp.tanh(gi[:, 2 * _H:] + r * gh[:, 2 * _H:])
    hn = (1.0 - z) * n + z * h
    if final_relu:
        hn = jax.nn.relu(hn)
    return hn


_GRU_SPECS = [pl.BlockSpec((1, _RB, _H), lambda i: (0, i, 0)),
              pl.BlockSpec((1, _RB, _H), lambda i: (1, i, 0)),
              pl.BlockSpec((_RB, _H), lambda i: (i, 0)),
              pl.BlockSpec((_H, 3 * _H), lambda i: (0, 0)),
              pl.BlockSpec((_H, 3 * _H), lambda i: (0, 0)),
              pl.BlockSpec((1, 3 * _H), lambda i: (0, 0)),
              pl.BlockSpec((1, 3 * _H), lambda i: (0, 0))]


def _make_gru_mm(final_relu):
    def body(p_ref, q_ref, h_ref, wih_ref, whh_ref, bih_ref, bhh_ref,
             wn_ref, h_out, m_out):
        hn = _gru_math(p_ref, q_ref, h_ref, wih_ref, whh_ref, bih_ref,
                       bhh_ref, final_relu)
        h_out[...] = hn
        m_out[...] = jnp.dot(hn, wn_ref[...],
                             preferred_element_type=jnp.float32)

    return pl.pallas_call(
        body,
        grid=(_NRB,),
        in_specs=_GRU_SPECS + [pl.BlockSpec((_H, _H), lambda i: (0, 0))],
        out_specs=[pl.BlockSpec((_RB, _H), lambda i: (i, 0)),
                   pl.BlockSpec((_RB, _H), lambda i: (i, 0))],
        out_shape=[jax.ShapeDtypeStruct((_N, _H), jnp.float32),
                   jax.ShapeDtypeStruct((_N, _H), jnp.float32)],
    )


_gru_mm = _make_gru_mm(False)
_gru_relu_mm = _make_gru_mm(True)


def _gru_out_body(p_ref, q_ref, h_ref, wih_ref, whh_ref, bih_ref, bhh_ref,
                  wo_ref, bo_ref, o_ref):
    hn = _gru_math(p_ref, q_ref, h_ref, wih_ref, whh_ref, bih_ref, bhh_ref,
                   True)
    o_ref[...] = (jnp.dot(hn, wo_ref[...], preferred_element_type=jnp.float32)
                  + bo_ref[...])


_gru_out = pl.pallas_call(
    _gru_out_body,
    grid=(_NRB,),
    in_specs=_GRU_SPECS + [pl.BlockSpec((_H, _NCLS), lambda i: (0, 0)),
                           pl.BlockSpec((1, _NCLS), lambda i: (0, 0))],
    out_specs=pl.BlockSpec((_RB, _NCLS), lambda i: (i, 0)),
    out_shape=jax.ShapeDtypeStruct((_N, _NCLS), jnp.float32),
)


def kernel(x, edge_index, edge_weight, Wi, bi,
           conv0_weight, conv0_Wih, conv0_Whh, conv0_bih, conv0_bhh,
           conv1_weight, conv1_Wih, conv1_Whh, conv1_bih, conv1_bhh,
           Wo, bo):
    src, dst, edge_weight = _prep_edges(edge_index[0], edge_index[1],
                                        edge_weight)
    convs = ((conv0_weight, conv0_Wih.T, conv0_Whh.T,
              conv0_bih.reshape(1, 3 * _H), conv0_bhh.reshape(1, 3 * _H)),
             (conv1_weight, conv1_Wih.T, conv1_Whh.T,
              conv1_bih.reshape(1, 3 * _H), conv1_bhh.reshape(1, 3 * _H)))
    h, m = _proj_mm(x, Wi, bi.reshape(1, _H), conv0_weight[0])
    for r in range(6):
        layer, i = divmod(r, 3)
        w3, wih_t, whh_t, bih2, bhh2 = convs[layer]
        parts = _propagate(m, src, dst, edge_weight)
        if r == 5:
            return _gru_out(parts, parts, h, wih_t, whh_t, bih2, bhh2,
                            Wo, bo.reshape(1, _NCLS))
        if i == 2:
            w_next = convs[layer + 1][0][0]
            h, m = _gru_relu_mm(parts, parts, h, wih_t, whh_t, bih2, bhh2,
                                w_next)
        else:
            w_next = w3[i + 1]
            h, m = _gru_mm(parts, parts, h, wih_t, whh_t, bih2, bhh2, w_next)


# speedup vs baseline: 1.3205x; 1.3205x over previous
import functools

import jax
import jax.numpy as jnp
from jax import lax
from jax.experimental import pallas as pl
from jax.experimental.pallas import tpu as pltpu
from jax.experimental.pallas import tpu_sc as plsc

_N = 10000
_E = 320000
_DF = 128
_H = 64
_NCLS = 40

_NSC = 2
_NTILE = 16
_NW = _NSC * _NTILE
_K = 80
_NCHUNK = 125
_EP = _NCHUNK * _K
_EPAD = _NW * _EP
_NPAD = 10240

_RB = 1000
_NRB = _N // _RB


_NBUF = 5
_GAHEAD = 2


def _sc_propagate_body(m_hbm, src_hbm, dst_hbm, w_hbm, out_hbm,
                       src_big, dst_big, w_big, rows_v, acc_sh,
                       gs0, gs1, gs2, gs3, gs4, ss0, ss1, ss2, ss3, ss4):
    gsem = (gs0, gs1, gs2, gs3, gs4)
    ssem = (ss0, ss1, ss2, ss3, ss4)
    c = lax.axis_index("c")
    s = lax.axis_index("s")
    wid = s * _NSC + c

    zero16 = jnp.zeros((16,), jnp.float32)
    for i in range(_K):
        for j in range(_H // 16):
            rows_v[0, i, pl.ds(j * 16, 16)] = zero16
    for z in range(_NPAD // (_NTILE * _K)):
        pltpu.sync_copy(rows_v.at[0],
                        acc_sh.at[pl.ds((z * _NTILE + s) * _K, _K)])

    pltpu.sync_copy(src_hbm.at[wid], src_big)
    pltpu.sync_copy(dst_hbm.at[wid], dst_big)
    pltpu.sync_copy(w_hbm.at[wid], w_big)
    plsc.subcore_barrier()

    def gather(cn, bn):
        pltpu.async_copy(m_hbm.at[src_big.at[cn]], rows_v.at[bn], gsem[bn])

    def wait_gather(cc, b):
        pltpu.make_async_copy(m_hbm.at[src_big.at[cc]], rows_v.at[b],
                              gsem[b]).wait()

    def scatter_start(cc, b):
        pltpu.async_copy(rows_v.at[b], acc_sh.at[dst_big.at[cc]], ssem[b],
                         add=True)

    def wait_scatter(cc, b):
        pltpu.make_async_copy(rows_v.at[b], acc_sh.at[dst_big.at[cc]],
                              ssem[b]).wait()

    def compute(b, cc):
        def grp(g, carry):
            wv = w_big[cc, pl.ds(g * 16, 16)]
            for e16 in range(16):
                wgt = wv[e16]
                e = g * 16 + e16
                for j in range(_H // 16):
                    sl = pl.ds(j * 16, 16)
                    rows_v[b, e, sl] = rows_v[b, e, sl] * wgt
            return carry
        lax.fori_loop(0, _K // 16, grp, 0)

    for b in range(_GAHEAD):
        gather(b, b)

    def quad(t, carry):
        for b in range(_NBUF):
            cc = t * _NBUF + b
            wait_gather(cc, b)
            compute(b, cc)
            scatter_start(cc, b)
            cn = cc + _GAHEAD
            bn = (b + _GAHEAD) % _NBUF

            @pl.when(cn < _NCHUNK)
            def _():
                @pl.when(cn >= _NBUF)
                def _():
                    wait_scatter(cn - _NBUF, bn)
                gather(cn, bn)
        return carry

    lax.fori_loop(0, _NCHUNK // _NBUF, quad, 0)

    for b in range(_NBUF):
        wait_scatter(_NCHUNK - _NBUF + b, b)
    plsc.subcore_barrier()

    for z in range(_NPAD // (_NTILE * _K)):
        base = (z * _NTILE + s) * _K
        pltpu.sync_copy(acc_sh.at[pl.ds(base, _K)],
                        out_hbm.at[c, pl.ds(base, _K)])


@functools.lru_cache(maxsize=None)
def _get_sc_propagate():
    return pl.kernel(
        _sc_propagate_body,
        out_type=jax.ShapeDtypeStruct((_NSC, _NPAD, _H), jnp.float32),
        mesh=plsc.VectorSubcoreMesh(core_axis_name="c", subcore_axis_name="s"),
        compiler_params=pltpu.CompilerParams(use_tc_tiling_on_sc=False),
        scratch_types=(
            [pltpu.VMEM((_NCHUNK, _K), jnp.int32),
             pltpu.VMEM((_NCHUNK, _K), jnp.int32),
             pltpu.VMEM((_NCHUNK, _K), jnp.float32),
             pltpu.VMEM((_NBUF, _K, _H), jnp.float32),
             pltpu.VMEM_SHARED((_NPAD, _H), jnp.float32)]
            + [pltpu.SemaphoreType.DMA] * (2 * _NBUF)
        ),
    )


def _prep_edges(src, dst, w):
    pad = _EPAD - _E
    return (jnp.pad(src, (0, pad)).reshape(_NW, _NCHUNK, _K),
            jnp.pad(dst, (0, pad)).reshape(_NW, _NCHUNK, _K),
            jnp.pad(w, (0, pad)).reshape(_NW, _NCHUNK, _K))


def _propagate(m, src3, dst3, w3):
    return _get_sc_propagate()(m, src3, dst3, w3)


def _proj_mm_body(x_ref, wi_ref, bi_ref, w0_ref, h_ref, m_ref):
    h = jax.nn.relu(
        jnp.dot(x_ref[...], wi_ref[...], preferred_element_type=jnp.float32)
        + bi_ref[...])
    h_ref[...] = h
    m_ref[...] = jnp.dot(h, w0_ref[...], preferred_element_type=jnp.float32)


_proj_mm = pl.pallas_call(
    _proj_mm_body,
    grid=(_NRB,),
    in_specs=[pl.BlockSpec((_RB, _DF), lambda i: (i, 0)),
              pl.BlockSpec((_DF, _H), lambda i: (0, 0)),
              pl.BlockSpec((1, _H), lambda i: (0, 0)),
              pl.BlockSpec((_H, _H), lambda i: (0, 0))],
    out_specs=[pl.BlockSpec((_RB, _H), lambda i: (i, 0)),
               pl.BlockSpec((_RB, _H), lambda i: (i, 0))],
    out_shape=[jax.ShapeDtypeStruct((_N, _H), jnp.float32),
               jax.ShapeDtypeStruct((_N, _H), jnp.float32)],
)


def _gru_math(p_ref, q_ref, h_ref, wih_ref, whh_ref, bih_ref, bhh_ref,
              final_relu):
    h = h_ref[...]
    agg = p_ref[0] + q_ref[0]
    gi = jnp.dot(agg, wih_ref[...],
                 preferred_element_type=jnp.float32) + bih_ref[...]
    gh = jnp.dot(h, whh_ref[...],
                 preferred_element_type=jnp.float32) + bhh_ref[...]
    r = jax.nn.sigmoid(gi[:, 0:_H] + gh[:, 0:_H])
    z = jax.nn.sigmoid(gi[:, _H:2 * _H] + gh[:, _H:2 * _H])
    n = jnp.tanh(gi[:, 2 * _H:] + r * gh[:, 2 * _H:])
    hn = (1.0 - z) * n + z * h
    if final_relu:
        hn = jax.nn.relu(hn)
    return hn


_GRU_SPECS = [pl.BlockSpec((1, _RB, _H), lambda i: (0, i, 0)),
              pl.BlockSpec((1, _RB, _H), lambda i: (1, i, 0)),
              pl.BlockSpec((_RB, _H), lambda i: (i, 0)),
              pl.BlockSpec((_H, 3 * _H), lambda i: (0, 0)),
              pl.BlockSpec((_H, 3 * _H), lambda i: (0, 0)),
              pl.BlockSpec((1, 3 * _H), lambda i: (0, 0)),
              pl.BlockSpec((1, 3 * _H), lambda i: (0, 0))]


def _make_gru_mm(final_relu):
    def body(p_ref, q_ref, h_ref, wih_ref, whh_ref, bih_ref, bhh_ref,
             wn_ref, h_out, m_out):
        hn = _gru_math(p_ref, q_ref, h_ref, wih_ref, whh_ref, bih_ref,
                       bhh_ref, final_relu)
        h_out[...] = hn
        m_out[...] = jnp.dot(hn, wn_ref[...],
                             preferred_element_type=jnp.float32)

    return pl.pallas_call(
        body,
        grid=(_NRB,),
        in_specs=_GRU_SPECS + [pl.BlockSpec((_H, _H), lambda i: (0, 0))],
        out_specs=[pl.BlockSpec((_RB, _H), lambda i: (i, 0)),
                   pl.BlockSpec((_RB, _H), lambda i: (i, 0))],
        out_shape=[jax.ShapeDtypeStruct((_N, _H), jnp.float32),
                   jax.ShapeDtypeStruct((_N, _H), jnp.float32)],
    )


_gru_mm = _make_gru_mm(False)
_gru_relu_mm = _make_gru_mm(True)


def _gru_out_body(p_ref, q_ref, h_ref, wih_ref, whh_ref, bih_ref, bhh_ref,
                  wo_ref, bo_ref, o_ref):
    hn = _gru_math(p_ref, q_ref, h_ref, wih_ref, whh_ref, bih_ref, bhh_ref,
                   True)
    o_ref[...] = (jnp.dot(hn, wo_ref[...], preferred_element_type=jnp.float32)
                  + bo_ref[...])


_gru_out = pl.pallas_call(
    _gru_out_body,
    grid=(_NRB,),
    in_specs=_GRU_SPECS + [pl.BlockSpec((_H, _NCLS), lambda i: (0, 0)),
                           pl.BlockSpec((1, _NCLS), lambda i: (0, 0))],
    out_specs=pl.BlockSpec((_RB, _NCLS), lambda i: (i, 0)),
    out_shape=jax.ShapeDtypeStruct((_N, _NCLS), jnp.float32),
)


def kernel(x, edge_index, edge_weight, Wi, bi,
           conv0_weight, conv0_Wih, conv0_Whh, conv0_bih, conv0_bhh,
           conv1_weight, conv1_Wih, conv1_Whh, conv1_bih, conv1_bhh,
           Wo, bo):
    src, dst, edge_weight = _prep_edges(edge_index[0], edge_index[1],
                                        edge_weight)
    convs = ((conv0_weight, conv0_Wih.T, conv0_Whh.T,
              conv0_bih.reshape(1, 3 * _H), conv0_bhh.reshape(1, 3 * _H)),
             (conv1_weight, conv1_Wih.T, conv1_Whh.T,
              conv1_bih.reshape(1, 3 * _H), conv1_bhh.reshape(1, 3 * _H)))
    h, m = _proj_mm(x, Wi, bi.reshape(1, _H), conv0_weight[0])
    for r in range(6):
        layer, i = divmod(r, 3)
        w3, wih_t, whh_t, bih2, bhh2 = convs[layer]
        parts = _propagate(m, src, dst, edge_weight)
        if r == 5:
            return _gru_out(parts, parts, h, wih_t, whh_t, bih2, bhh2,
                            Wo, bo.reshape(1, _NCLS))
        if i == 2:
            w_next = convs[layer + 1][0][0]
            h, m = _gru_relu_mm(parts, parts, h, wih_t, whh_t, bih2, bhh2,
                                w_next)
        else:
            w_next = w3[i + 1]
            h, m = _gru_mm(parts, parts, h, wih_t, whh_t, bih2, bhh2, w_next)


# speedup vs baseline: 1.7171x; 1.3003x over previous
import functools

import jax
import jax.numpy as jnp
from jax import lax
from jax.experimental import pallas as pl
from jax.experimental.pallas import tpu as pltpu
from jax.experimental.pallas import tpu_sc as plsc

_N = 10000
_E = 320000
_DF = 128
_H = 64
_NCLS = 40

_NSC = 2
_NTILE = 16
_NW = _NSC * _NTILE
_K = 40
_NCHUNK = 250
_EP = _NCHUNK * _K
_EPAD = _NW * _EP
_NPAD = 10240

_RB = 1000
_NRB = _N // _RB


_NBUF = 5
_GAHEAD = 2


def _sc_propagate_body(m_hbm, src_hbm, dst_hbm, w_hbm, out_hbm,
                       src_big, dst_big, w_big, rows_v, acc_sh,
                       gs0, gs1, gs2, gs3, gs4, ss0, ss1, ss2, ss3, ss4):
    gsem = (gs0, gs1, gs2, gs3, gs4)
    ssem = (ss0, ss1, ss2, ss3, ss4)
    c = lax.axis_index("c")
    s = lax.axis_index("s")
    wid = s * _NSC + c

    zero16 = jnp.zeros((16,), jnp.float32)
    for i in range(_K):
        for j in range(_H // 16):
            rows_v[0, i, pl.ds(j * 16, 16)] = zero16
    for z in range(_NPAD // (_NTILE * _K)):
        pltpu.sync_copy(rows_v.at[0],
                        acc_sh.at[pl.ds((z * _NTILE + s) * _K, _K)])

    pltpu.sync_copy(src_hbm.at[wid], src_big)
    pltpu.sync_copy(dst_hbm.at[wid], dst_big)
    pltpu.sync_copy(w_hbm.at[wid], w_big)
    plsc.subcore_barrier()

    def gather(cn, bn):
        pltpu.async_copy(m_hbm.at[src_big.at[cn]], rows_v.at[bn], gsem[bn])

    def wait_gather(cc, b):
        pltpu.make_async_copy(m_hbm.at[src_big.at[cc]], rows_v.at[b],
                              gsem[b]).wait()

    def scatter_start(cc, b):
        pltpu.async_copy(rows_v.at[b], acc_sh.at[dst_big.at[cc]], ssem[b],
                         add=True)

    def wait_scatter(cc, b):
        pltpu.make_async_copy(rows_v.at[b], acc_sh.at[dst_big.at[cc]],
                              ssem[b]).wait()

    def compute(b, cc):
        def grp(g, carry):
            wv = w_big[cc, pl.ds(g * 16, 16)]
            for e16 in range(16):
                wgt = wv[e16]
                e = g * 16 + e16
                for j in range(_H // 16):
                    sl = pl.ds(j * 16, 16)
                    rows_v[b, e, sl] = rows_v[b, e, sl] * wgt
            return carry
        lax.fori_loop(0, _K // 16, grp, 0)

    for b in range(_GAHEAD):
        gather(b, b)

    def quad(t, carry):
        for b in range(_NBUF):
            cc = t * _NBUF + b
            wait_gather(cc, b)
            compute(b, cc)
            scatter_start(cc, b)
            cn = cc + _GAHEAD
            bn = (b + _GAHEAD) % _NBUF

            @pl.when(cn < _NCHUNK)
            def _():
                @pl.when(cn >= _NBUF)
                def _():
                    wait_scatter(cn - _NBUF, bn)
                gather(cn, bn)
        return carry

    lax.fori_loop(0, _NCHUNK // _NBUF, quad, 0)

    for b in range(_NBUF):
        wait_scatter(_NCHUNK - _NBUF + b, b)
    plsc.subcore_barrier()

    for z in range(_NPAD // (_NTILE * _K)):
        base = (z * _NTILE + s) * _K
        pltpu.sync_copy(acc_sh.at[pl.ds(base, _K)],
                        out_hbm.at[c, pl.ds(base, _K)])


@functools.lru_cache(maxsize=None)
def _get_sc_propagate():
    return pl.kernel(
        _sc_propagate_body,
        out_type=jax.ShapeDtypeStruct((_NSC, _NPAD, _H), jnp.float32),
        mesh=plsc.VectorSubcoreMesh(core_axis_name="c", subcore_axis_name="s"),
        compiler_params=pltpu.CompilerParams(use_tc_tiling_on_sc=False),
        scratch_types=(
            [pltpu.VMEM((_NCHUNK, _K), jnp.int32),
             pltpu.VMEM((_NCHUNK, _K), jnp.int32),
             pltpu.VMEM((_NCHUNK, _K), jnp.float32),
             pltpu.VMEM((_NBUF, _K, _H), jnp.float32),
             pltpu.VMEM_SHARED((_NPAD, _H), jnp.float32)]
            + [pltpu.SemaphoreType.DMA] * (2 * _NBUF)
        ),
    )


def _prep_edges(src, dst, w):
    pad = _EPAD - _E
    return (jnp.pad(src, (0, pad)).reshape(_NW, _NCHUNK, _K),
            jnp.pad(dst, (0, pad)).reshape(_NW, _NCHUNK, _K),
            jnp.pad(w, (0, pad)).reshape(_NW, _NCHUNK, _K))


def _propagate(m, src3, dst3, w3):
    return _get_sc_propagate()(m, src3, dst3, w3)


def _proj_mm_body(x_ref, wi_ref, bi_ref, w0_ref, h_ref, m_ref):
    h = jax.nn.relu(
        jnp.dot(x_ref[...], wi_ref[...], preferred_element_type=jnp.float32)
        + bi_ref[...])
    h_ref[...] = h
    m_ref[...] = jnp.dot(h, w0_ref[...], preferred_element_type=jnp.float32)


_proj_mm = pl.pallas_call(
    _proj_mm_body,
    grid=(_NRB,),
    in_specs=[pl.BlockSpec((_RB, _DF), lambda i: (i, 0)),
              pl.BlockSpec((_DF, _H), lambda i: (0, 0)),
              pl.BlockSpec((1, _H), lambda i: (0, 0)),
              pl.BlockSpec((_H, _H), lambda i: (0, 0))],
    out_specs=[pl.BlockSpec((_RB, _H), lambda i: (i, 0)),
               pl.BlockSpec((_RB, _H), lambda i: (i, 0))],
    out_shape=[jax.ShapeDtypeStruct((_N, _H), jnp.float32),
               jax.ShapeDtypeStruct((_N, _H), jnp.float32)],
)


def _gru_math(p_ref, q_ref, h_ref, wih_ref, whh_ref, bih_ref, bhh_ref,
              final_relu):
    h = h_ref[...]
    agg = p_ref[0] + q_ref[0]
    gi = jnp.dot(agg, wih_ref[...],
                 preferred_element_type=jnp.float32) + bih_ref[...]
    gh = jnp.dot(h, whh_ref[...],
                 preferred_element_type=jnp.float32) + bhh_ref[...]
    r = jax.nn.sigmoid(gi[:, 0:_H] + gh[:, 0:_H])
    z = jax.nn.sigmoid(gi[:, _H:2 * _H] + gh[:, _H:2 * _H])
    n = jnp.tanh(gi[:, 2 * _H:] + r * gh[:, 2 * _H:])
    hn = (1.0 - z) * n + z * h
    if final_relu:
        hn = jax.nn.relu(hn)
    return hn


_GRU_SPECS = [pl.BlockSpec((1, _RB, _H), lambda i: (0, i, 0)),
              pl.BlockSpec((1, _RB, _H), lambda i: (1, i, 0)),
              pl.BlockSpec((_RB, _H), lambda i: (i, 0)),
              pl.BlockSpec((_H, 3 * _H), lambda i: (0, 0)),
              pl.BlockSpec((_H, 3 * _H), lambda i: (0, 0)),
              pl.BlockSpec((1, 3 * _H), lambda i: (0, 0)),
              pl.BlockSpec((1, 3 * _H), lambda i: (0, 0))]


def _make_gru_mm(final_relu):
    def body(p_ref, q_ref, h_ref, wih_ref, whh_ref, bih_ref, bhh_ref,
             wn_ref, h_out, m_out):
        hn = _gru_math(p_ref, q_ref, h_ref, wih_ref, whh_ref, bih_ref,
                       bhh_ref, final_relu)
        h_out[...] = hn
        m_out[...] = jnp.dot(hn, wn_ref[...],
                             preferred_element_type=jnp.float32)

    return pl.pallas_call(
        body,
        grid=(_NRB,),
        in_specs=_GRU_SPECS + [pl.BlockSpec((_H, _H), lambda i: (0, 0))],
        out_specs=[pl.BlockSpec((_RB, _H), lambda i: (i, 0)),
                   pl.BlockSpec((_RB, _H), lambda i: (i, 0))],
        out_shape=[jax.ShapeDtypeStruct((_N, _H), jnp.float32),
                   jax.ShapeDtypeStruct((_N, _H), jnp.float32)],
    )


_gru_mm = _make_gru_mm(False)
_gru_relu_mm = _make_gru_mm(True)


def _gru_out_body(p_ref, q_ref, h_ref, wih_ref, whh_ref, bih_ref, bhh_ref,
                  wo_ref, bo_ref, o_ref):
    hn = _gru_math(p_ref, q_ref, h_ref, wih_ref, whh_ref, bih_ref, bhh_ref,
                   True)
    o_ref[...] = (jnp.dot(hn, wo_ref[...], preferred_element_type=jnp.float32)
                  + bo_ref[...])


_gru_out = pl.pallas_call(
    _gru_out_body,
    grid=(_NRB,),
    in_specs=_GRU_SPECS + [pl.BlockSpec((_H, _NCLS), lambda i: (0, 0)),
                           pl.BlockSpec((1, _NCLS), lambda i: (0, 0))],
    out_specs=pl.BlockSpec((_RB, _NCLS), lambda i: (i, 0)),
    out_shape=jax.ShapeDtypeStruct((_N, _NCLS), jnp.float32),
)


def kernel(x, edge_index, edge_weight, Wi, bi,
           conv0_weight, conv0_Wih, conv0_Whh, conv0_bih, conv0_bhh,
           conv1_weight, conv1_Wih, conv1_Whh, conv1_bih, conv1_bhh,
           Wo, bo):
    src, dst, edge_weight = _prep_edges(edge_index[0], edge_index[1],
                                        edge_weight)
    convs = ((conv0_weight, conv0_Wih.T, conv0_Whh.T,
              conv0_bih.reshape(1, 3 * _H), conv0_bhh.reshape(1, 3 * _H)),
             (conv1_weight, conv1_Wih.T, conv1_Whh.T,
              conv1_bih.reshape(1, 3 * _H), conv1_bhh.reshape(1, 3 * _H)))
    h, m = _proj_mm(x, Wi, bi.reshape(1, _H), conv0_weight[0])
    for r in range(6):
        layer, i = divmod(r, 3)
        w3, wih_t, whh_t, bih2, bhh2 = convs[layer]
        parts = _propagate(m, src, dst, edge_weight)
        if r == 5:
            return _gru_out(parts, parts, h, wih_t, whh_t, bih2, bhh2,
                            Wo, bo.reshape(1, _NCLS))
        if i == 2:
            w_next = convs[layer + 1][0][0]
            h, m = _gru_relu_mm(parts, parts, h, wih_t, whh_t, bih2, bhh2,
                                w_next)
        else:
            w_next = w3[i + 1]
            h, m = _gru_mm(parts, parts, h, wih_t, whh_t, bih2, bhh2, w_next)
